# Initial kernel scaffold; baseline (speedup 1.0000x reference)
import functools

import jax
import jax.numpy as jnp
from jax import lax
from jax.experimental import pallas as pl
from jax.experimental.pallas import tpu as pltpu
from jax.experimental.pallas import tpu_sc as plsc

B, H, W, SH, SW = 2, 64, 256, 32, 128
KH, KW = 4, 8
NSAMPLE = 8
DIST = 100.0
C1, C2 = 64, 64
NQ = B * H * W
NTILES = 32
QT = NQ // NTILES
NGROUPS = QT // 16
TROWS = B * SH * SW
ZROW = TROWS
TPAD = 8200
GC = 80
EPS = 1e-5
NROWS = NSAMPLE * NQ


def _min_tree(vals):
    while len(vals) > 1:
        nxt = [jnp.minimum(vals[2 * i], vals[2 * i + 1])
               for i in range(len(vals) // 2)]
        if len(vals) % 2:
            nxt.append(vals[-1])
        vals = nxt
    return vals[0]


def _sc_select_gather(x2p, x1r, table):
    mesh = plsc.VectorSubcoreMesh(core_axis_name="c", subcore_axis_name="s")

    @functools.partial(
        pl.kernel,
        out_type=jax.ShapeDtypeStruct((NROWS, GC), jnp.float32),
        mesh=mesh,
        scratch_types=[
            pltpu.VMEM((TPAD,), jnp.float32),
            pltpu.VMEM((TPAD,), jnp.float32),
            pltpu.VMEM((TPAD,), jnp.float32),
            pltpu.VMEM((QT,), jnp.float32),
            pltpu.VMEM((QT,), jnp.float32),
            pltpu.VMEM((QT,), jnp.float32),
            pltpu.VMEM((NSAMPLE * QT,), jnp.int32),
            pltpu.VMEM((QT, GC), jnp.float32),
            pltpu.SemaphoreType.DMA,
        ],
    )
    def body(x2p_hbm, x1r_hbm, tab_hbm, out_hbm,
             x2x, x2y, x2z, x1x, x1y, x1z, idxv, gbuf, sem):
        cid = lax.axis_index("c")
        sid = lax.axis_index("s")
        wid = sid * 2 + cid
        base = wid * QT
        pltpu.sync_copy(x2p_hbm.at[0], x2x)
        pltpu.sync_copy(x2p_hbm.at[1], x2y)
        pltpu.sync_copy(x2p_hbm.at[2], x2z)
        pltpu.sync_copy(x1r_hbm.at[wid, 0], x1x)
        pltpu.sync_copy(x1r_hbm.at[wid, 1], x1y)
        pltpu.sync_copy(x1r_hbm.at[wid, 2], x1z)

        lanes = lax.iota(jnp.int32, 16)
        sentinel = jnp.int32(0x7FFFFFFF)

        def g_body(g, carry):
            off = g * 16
            q = base + off + lanes
            wq = q & 255
            hq = (q >> 8) & 63
            bq = q >> 14
            h2 = hq >> 1
            w2 = wq >> 1
            x1 = x1x[pl.ds(off, 16)]
            y1 = x1y[pl.ds(off, 16)]
            z1 = x1z[pl.ds(off, 16)]
            keys = []
            for kh in range(KH):
                r = h2 + (kh - KH // 2)
                vr = (r >= 0) & (r < SH)
                rc = jnp.clip(r, 0, SH - 1)
                rbase = (bq << 12) + (rc << 7)
                for kw in range(KW):
                    c = w2 + (kw - KW // 2)
                    v = vr & (c >= 0) & (c < SW)
                    cc = jnp.clip(c, 0, SW - 1)
                    fl = rbase + cc
                    cx = plsc.load_gather(x2x, [fl])
                    cy = plsc.load_gather(x2y, [fl])
                    cz = plsc.load_gather(x2z, [fl])
                    dx = cx - x1
                    dy = cy - y1
                    dz = cz - z1
                    d2 = dx * dx + dy * dy + dz * dz
                    key = (plsc.bitcast(d2, jnp.int32) & jnp.int32(-32)) | (kh * KW + kw)
                    keys.append(jnp.where(v, key, sentinel))
            for s in range(NSAMPLE):
                m = _min_tree(keys)
                ksel = m & 31
                d2s = plsc.bitcast(m & jnp.int32(-32), jnp.float32)
                ok = d2s < jnp.float32(DIST * DIST)
                khs = ksel >> 3
                kws = ksel & 7
                rs = jnp.clip(h2 + khs - KH // 2, 0, SH - 1)
                cs = jnp.clip(w2 + kws - KW // 2, 0, SW - 1)
                fl = (bq << 12) + (rs << 7) + cs
                idxv[pl.ds(s * QT + off, 16)] = jnp.where(ok, fl, jnp.int32(ZROW))
                if s < NSAMPLE - 1:
                    keys = [jnp.where(k == m, sentinel, k) for k in keys]
            return carry

        lax.fori_loop(0, NGROUPS, g_body, 0)

        def s_body(s, carry):
            for ch in range(QT // 128):
                idxsl = idxv.at[pl.ds(s * QT + ch * 128, 128)]
                pltpu.async_copy(tab_hbm.at[idxsl],
                                 gbuf.at[pl.ds(ch * 128, 128)], sem).wait()
            pltpu.sync_copy(gbuf, out_hbm.at[pl.ds(s * NQ + base, QT)])
            return carry

        lax.fori_loop(0, NSAMPLE, s_body, 0)

    return body(x2p, x1r, table)


def _stats_accum(st_ref, z, first):
    @pl.when(first)
    def _():
        st_ref[...] = jnp.zeros_like(st_ref)
    st_ref[0:1, :] += jnp.sum(z, axis=0, keepdims=True)
    st_ref[1:2, :] += jnp.sum(z * z, axis=0, keepdims=True)


def _bn_coeffs(st_ref, g_ref, be_ref, n):
    mu = st_ref[0:1, :] / n
    var = st_ref[1:2, :] / n - mu * mu
    inv = lax.rsqrt(var + EPS)
    scale = inv * g_ref[...]
    shift = be_ref[...] - mu * scale
    return scale, shift


def _layer0(gmat, x1p, w0cat, w0a8, b0):
    CQ = 2048
    grid = (NSAMPLE, NQ // CQ)

    def body(g_ref, x1_ref, wc_ref, wa_ref, b_ref, z_ref, st_ref):
        z = jnp.dot(g_ref[0], wc_ref[...], preferred_element_type=jnp.float32)
        z = z - jnp.dot(x1_ref[...], wa_ref[...], preferred_element_type=jnp.float32)
        z = z + b_ref[...]
        z_ref[...] = z
        i, j = pl.program_id(0), pl.program_id(1)
        _stats_accum(st_ref, z, (i == 0) & (j == 0))

    return pl.pallas_call(
        body,
        grid=grid,
        in_specs=[
            pl.BlockSpec((1, CQ, GC), lambda i, j: (i, j, 0)),
            pl.BlockSpec((CQ, 8), lambda i, j: (j, 0)),
            pl.BlockSpec((GC, 128), lambda i, j: (0, 0)),
            pl.BlockSpec((8, 128), lambda i, j: (0, 0)),
            pl.BlockSpec((1, 128), lambda i, j: (0, 0)),
        ],
        out_specs=[
            pl.BlockSpec((CQ, 128), lambda i, j: (i * (NQ // CQ) + j, 0)),
            pl.BlockSpec((8, 128), lambda i, j: (0, 0)),
        ],
        out_shape=[
            jax.ShapeDtypeStruct((NROWS, 128), jnp.float32),
            jax.ShapeDtypeStruct((8, 128), jnp.float32),
        ],
    )(gmat.reshape(NSAMPLE, NQ, GC), x1p, w0cat, w0a8, b0)


def _layer_mid(z, st, g, be, wm, bm, cout):
    CR = 4096
    cin = z.shape[1]
    grid = (NROWS // CR,)

    def body(z_ref, st_ref, g_ref, be_ref, w_ref, b_ref, z2_ref, st2_ref):
        scale, shift = _bn_coeffs(st_ref, g_ref, be_ref, float(NROWS))
        a = jnp.maximum(z_ref[...] * scale + shift, 0.0)
        z2 = jnp.dot(a, w_ref[...], preferred_element_type=jnp.float32) + b_ref[...]
        z2_ref[...] = z2
        _stats_accum(st2_ref, z2, pl.program_id(0) == 0)

    return pl.pallas_call(
        body,
        grid=grid,
        in_specs=[
            pl.BlockSpec((CR, cin), lambda i: (i, 0)),
            pl.BlockSpec((8, cin), lambda i: (0, 0)),
            pl.BlockSpec((1, cin), lambda i: (0, 0)),
            pl.BlockSpec((1, cin), lambda i: (0, 0)),
            pl.BlockSpec((cin, cout), lambda i: (0, 0)),
            pl.BlockSpec((1, cout), lambda i: (0, 0)),
        ],
        out_specs=[
            pl.BlockSpec((CR, cout), lambda i: (i, 0)),
            pl.BlockSpec((8, cout), lambda i: (0, 0)),
        ],
        out_shape=[
            jax.ShapeDtypeStruct((NROWS, cout), jnp.float32),
            jax.ShapeDtypeStruct((8, cout), jnp.float32),
        ],
    )(z, st, g, be, wm, bm)


def _layer_pool(z3, st3, g2, be2, p1, w3m, w3p, b3):
    CQ = 1024
    grid = (NQ // CQ,)

    def body(z_ref, st_ref, g_ref, be_ref, p_ref, wm_ref, wp_ref, b_ref,
             z4_ref, st4_ref):
        scale, shift = _bn_coeffs(st_ref, g_ref, be_ref, float(NROWS))
        a = jnp.maximum(z_ref[...] * scale + shift, 0.0)
        m = jnp.max(a, axis=0)
        z4 = jnp.dot(m, wm_ref[...], preferred_element_type=jnp.float32)
        z4 = z4 + jnp.dot(p_ref[...], wp_ref[...], preferred_element_type=jnp.float32)
        z4 = z4 + b_ref[...]
        z4_ref[...] = z4
        _stats_accum(st4_ref, z4, pl.program_id(0) == 0)

    return pl.pallas_call(
        body,
        grid=grid,
        in_specs=[
            pl.BlockSpec((NSAMPLE, CQ, 256), lambda i: (0, i, 0)),
            pl.BlockSpec((8, 256), lambda i: (0, 0)),
            pl.BlockSpec((1, 256), lambda i: (0, 0)),
            pl.BlockSpec((1, 256), lambda i: (0, 0)),
            pl.BlockSpec((CQ, C1), lambda i: (i, 0)),
            pl.BlockSpec((256, 256), lambda i: (0, 0)),
            pl.BlockSpec((C1, 256), lambda i: (0, 0)),
            pl.BlockSpec((1, 256), lambda i: (0, 0)),
        ],
        out_specs=[
            pl.BlockSpec((CQ, 256), lambda i: (i, 0)),
            pl.BlockSpec((8, 256), lambda i: (0, 0)),
        ],
        out_shape=[
            jax.ShapeDtypeStruct((NQ, 256), jnp.float32),
            jax.ShapeDtypeStruct((8, 256), jnp.float32),
        ],
    )(z3.reshape(NSAMPLE, NQ, 256), st3, g2, be2, p1, w3m, w3p, b3)


def _layer_out(z4, st4, g3, be3):
    CQ = 4096
    grid = (NQ // CQ,)

    def body(z_ref, st_ref, g_ref, be_ref, o_ref):
        scale, shift = _bn_coeffs(st_ref, g_ref, be_ref, float(NQ))
        o_ref[...] = jnp.maximum(z_ref[...] * scale + shift, 0.0)

    return pl.pallas_call(
        body,
        grid=grid,
        in_specs=[
            pl.BlockSpec((CQ, 256), lambda i: (i, 0)),
            pl.BlockSpec((8, 256), lambda i: (0, 0)),
            pl.BlockSpec((1, 256), lambda i: (0, 0)),
            pl.BlockSpec((1, 256), lambda i: (0, 0)),
        ],
        out_specs=pl.BlockSpec((CQ, 256), lambda i: (i, 0)),
        out_shape=jax.ShapeDtypeStruct((NQ, 256), jnp.float32),
    )(z4, st4, g3, be3)


def kernel(xyz1_proj, xyz2_proj, points1_proj, feat2_proj,
           W0, b0, g0, be0, W1, b1, g1, be1, W2, b2, g2, be2,
           W3, b3, g3, be3):
    f32 = jnp.float32
    t_xyz = xyz2_proj.reshape(TROWS, 3).astype(f32)
    t_feat = feat2_proj.reshape(TROWS, C2).astype(f32)
    table = jnp.zeros((TPAD, GC), f32)
    table = table.at[:TROWS, :3].set(t_xyz).at[:TROWS, 3:3 + C2].set(t_feat)
    x2p = jnp.zeros((3, TPAD), f32).at[:, :TROWS].set(t_xyz.T)
    x1flat = xyz1_proj.reshape(NQ, 3).astype(f32)
    x1r = x1flat.T.reshape(3, NTILES, QT).transpose(1, 0, 2)

    gmat = _sc_select_gather(x2p, x1r, table)

    w0cat = jnp.zeros((GC, 128), f32).at[:3 + C2, :].set(W0)
    w0a8 = jnp.zeros((8, 128), f32).at[:3, :].set(W0[:3])
    x1p = jnp.zeros((NQ, 8), f32).at[:, :3].set(x1flat)
    p1 = points1_proj.reshape(NQ, C1).astype(f32)

    z1, st1 = _layer0(gmat, x1p, w0cat, w0a8, b0.reshape(1, 128))
    z2, st2 = _layer_mid(z1, st1, g0.reshape(1, 128), be0.reshape(1, 128),
                         W1, b1.reshape(1, 128), 128)
    z3, st3 = _layer_mid(z2, st2, g1.reshape(1, 128), be1.reshape(1, 128),
                         W2, b2.reshape(1, 256), 256)
    z4, st4 = _layer_pool(z3, st3, g2.reshape(1, 256), be2.reshape(1, 256),
                          p1, W3[:256], W3[256:], b3.reshape(1, 256))
    out = _layer_out(z4, st4, g3.reshape(1, 256), be3.reshape(1, 256))
    return out.reshape(B, H * W, 256)


# speedup vs baseline: 16.3621x; 16.3621x over previous
import functools

import jax
import jax.numpy as jnp
from jax import lax
from jax.experimental import pallas as pl
from jax.experimental.pallas import tpu as pltpu
from jax.experimental.pallas import tpu_sc as plsc

B, H, W, SH, SW = 2, 64, 256, 32, 128
KH, KW = 4, 8
NSAMPLE = 8
DIST = 100.0
C1, C2 = 64, 64
NQ = B * H * W
NTILES = 32
QT = NQ // NTILES
NGROUPS = QT // 16
TROWS = B * SH * SW
ZROW = TROWS
TPAD = 8200
GC = 128
EPS = 1e-5
NROWS = NSAMPLE * NQ


def _min_tree(vals):
    while len(vals) > 1:
        nxt = [jnp.minimum(vals[2 * i], vals[2 * i + 1])
               for i in range(len(vals) // 2)]
        if len(vals) % 2:
            nxt.append(vals[-1])
        vals = nxt
    return vals[0]


def _sc_select_gather(x2planes, x1planes, table):
    mesh = plsc.VectorSubcoreMesh(core_axis_name="c", subcore_axis_name="s")

    @functools.partial(
        pl.kernel,
        out_type=jax.ShapeDtypeStruct((NROWS, GC), jnp.float32),
        mesh=mesh,
        compiler_params=pltpu.CompilerParams(needs_layout_passes=False),
        scratch_types=[
            pltpu.VMEM((TPAD,), jnp.float32),
            pltpu.VMEM((TPAD,), jnp.float32),
            pltpu.VMEM((TPAD,), jnp.float32),
            pltpu.VMEM((QT,), jnp.float32),
            pltpu.VMEM((QT,), jnp.float32),
            pltpu.VMEM((QT,), jnp.float32),
            pltpu.VMEM((NSAMPLE * QT,), jnp.int32),
            pltpu.VMEM((512, GC), jnp.float32),
            pltpu.SemaphoreType.DMA,
        ],
    )
    def body(x2x_h, x2y_h, x2z_h, x1x_h, x1y_h, x1z_h, tab_hbm, out_hbm,
             x2x, x2y, x2z, x1x, x1y, x1z, idxv, gbuf, sem):
        cid = lax.axis_index("c")
        sid = lax.axis_index("s")
        wid = sid * 2 + cid
        base = wid * QT
        pltpu.sync_copy(x2x_h, x2x)
        pltpu.sync_copy(x2y_h, x2y)
        pltpu.sync_copy(x2z_h, x2z)
        pltpu.sync_copy(x1x_h.at[pl.ds(base, QT)], x1x)
        pltpu.sync_copy(x1y_h.at[pl.ds(base, QT)], x1y)
        pltpu.sync_copy(x1z_h.at[pl.ds(base, QT)], x1z)

        lanes = lax.iota(jnp.int32, 16)
        sentinel = jnp.int32(0x7FFFFFFF)

        def g_body(g, carry):
            off = g * 16
            q = base + off + lanes
            wq = q & 255
            hq = (q >> 8) & 63
            bq = q >> 14
            h2 = hq >> 1
            w2 = wq >> 1
            x1 = x1x[pl.ds(off, 16)]
            y1 = x1y[pl.ds(off, 16)]
            z1 = x1z[pl.ds(off, 16)]
            keys = []
            for kh in range(KH):
                r = h2 + (kh - KH // 2)
                vr = (r >= 0) & (r < SH)
                rc = jnp.clip(r, 0, SH - 1)
                rbase = (bq << 12) + (rc << 7)
                for kw in range(KW):
                    c = w2 + (kw - KW // 2)
                    v = vr & (c >= 0) & (c < SW)
                    cc = jnp.clip(c, 0, SW - 1)
                    fl = rbase + cc
                    cx = plsc.load_gather(x2x, [fl])
                    cy = plsc.load_gather(x2y, [fl])
                    cz = plsc.load_gather(x2z, [fl])
                    dx = cx - x1
                    dy = cy - y1
                    dz = cz - z1
                    d2 = dx * dx + dy * dy + dz * dz
                    key = (plsc.bitcast(d2, jnp.int32) & jnp.int32(-32)) | (kh * KW + kw)
                    keys.append(jnp.where(v, key, sentinel))
            for s in range(NSAMPLE):
                m = _min_tree(keys)
                ksel = m & 31
                d2s = plsc.bitcast(m & jnp.int32(-32), jnp.float32)
                ok = d2s < jnp.float32(DIST * DIST)
                khs = ksel >> 3
                kws = ksel & 7
                rs = jnp.clip(h2 + khs - KH // 2, 0, SH - 1)
                cs = jnp.clip(w2 + kws - KW // 2, 0, SW - 1)
                fl = (bq << 12) + (rs << 7) + cs
                idxv[pl.ds(s * QT + off, 16)] = jnp.where(ok, fl, jnp.int32(ZROW))
                if s < NSAMPLE - 1:
                    keys = [jnp.where(k == m, sentinel, k) for k in keys]
            return carry

        lax.fori_loop(0, NGROUPS, g_body, 0)

        def s_body(j, carry):
            s = j >> 1
            half = j & 1
            for ch in range(4):
                idxsl = idxv.at[pl.ds(s * QT + half * 512 + ch * 128, 128)]
                pltpu.async_copy(tab_hbm.at[idxsl],
                                 gbuf.at[pl.ds(ch * 128, 128)], sem).wait()
            pltpu.sync_copy(gbuf,
                            out_hbm.at[pl.ds(s * NQ + base + half * 512, 512)])
            return carry

        lax.fori_loop(0, 2 * NSAMPLE, s_body, 0)

    return body(*x2planes, *x1planes, table)


def _stats_accum(st_ref, z, first):
    @pl.when(first)
    def _():
        st_ref[...] = jnp.zeros_like(st_ref)
    st_ref[0:1, :] += jnp.sum(z, axis=0, keepdims=True)
    st_ref[1:2, :] += jnp.sum(z * z, axis=0, keepdims=True)


def _bn_coeffs(st_ref, g_ref, be_ref, n):
    mu = st_ref[0:1, :] / n
    var = st_ref[1:2, :] / n - mu * mu
    inv = lax.rsqrt(var + EPS)
    scale = inv * g_ref[...]
    shift = be_ref[...] - mu * scale
    return scale, shift


def _layer0(gmat, x1p, w0cat, w0a8, b0):
    CQ = 2048
    grid = (NSAMPLE, NQ // CQ)

    def body(g_ref, x1_ref, wc_ref, wa_ref, b_ref, z_ref, st_ref):
        z = jnp.dot(g_ref[0], wc_ref[...], preferred_element_type=jnp.float32)
        z = z - jnp.dot(x1_ref[...], wa_ref[...], preferred_element_type=jnp.float32)
        z = z + b_ref[...]
        z_ref[...] = z
        i, j = pl.program_id(0), pl.program_id(1)
        _stats_accum(st_ref, z, (i == 0) & (j == 0))

    return pl.pallas_call(
        body,
        grid=grid,
        in_specs=[
            pl.BlockSpec((1, CQ, GC), lambda i, j: (i, j, 0)),
            pl.BlockSpec((CQ, 8), lambda i, j: (j, 0)),
            pl.BlockSpec((GC, 128), lambda i, j: (0, 0)),
            pl.BlockSpec((8, 128), lambda i, j: (0, 0)),
            pl.BlockSpec((1, 128), lambda i, j: (0, 0)),
        ],
        out_specs=[
            pl.BlockSpec((CQ, 128), lambda i, j: (i * (NQ // CQ) + j, 0)),
            pl.BlockSpec((8, 128), lambda i, j: (0, 0)),
        ],
        out_shape=[
            jax.ShapeDtypeStruct((NROWS, 128), jnp.float32),
            jax.ShapeDtypeStruct((8, 128), jnp.float32),
        ],
    )(gmat.reshape(NSAMPLE, NQ, GC), x1p, w0cat, w0a8, b0)


def _layer_mid(z, st, g, be, wm, bm, cout):
    CR = 4096
    cin = z.shape[1]
    grid = (NROWS // CR,)

    def body(z_ref, st_ref, g_ref, be_ref, w_ref, b_ref, z2_ref, st2_ref):
        scale, shift = _bn_coeffs(st_ref, g_ref, be_ref, float(NROWS))
        a = jnp.maximum(z_ref[...] * scale + shift, 0.0)
        z2 = jnp.dot(a, w_ref[...], preferred_element_type=jnp.float32) + b_ref[...]
        z2_ref[...] = z2
        _stats_accum(st2_ref, z2, pl.program_id(0) == 0)

    return pl.pallas_call(
        body,
        grid=grid,
        in_specs=[
            pl.BlockSpec((CR, cin), lambda i: (i, 0)),
            pl.BlockSpec((8, cin), lambda i: (0, 0)),
            pl.BlockSpec((1, cin), lambda i: (0, 0)),
            pl.BlockSpec((1, cin), lambda i: (0, 0)),
            pl.BlockSpec((cin, cout), lambda i: (0, 0)),
            pl.BlockSpec((1, cout), lambda i: (0, 0)),
        ],
        out_specs=[
            pl.BlockSpec((CR, cout), lambda i: (i, 0)),
            pl.BlockSpec((8, cout), lambda i: (0, 0)),
        ],
        out_shape=[
            jax.ShapeDtypeStruct((NROWS, cout), jnp.float32),
            jax.ShapeDtypeStruct((8, cout), jnp.float32),
        ],
    )(z, st, g, be, wm, bm)


def _layer_pool(z3, st3, g2, be2, p1, w3m, w3p, b3):
    CQ = 1024
    grid = (NQ // CQ,)

    def body(z_ref, st_ref, g_ref, be_ref, p_ref, wm_ref, wp_ref, b_ref,
             z4_ref, st4_ref):
        scale, shift = _bn_coeffs(st_ref, g_ref, be_ref, float(NROWS))
        a = jnp.maximum(z_ref[...] * scale + shift, 0.0)
        m = jnp.max(a, axis=0)
        z4 = jnp.dot(m, wm_ref[...], preferred_element_type=jnp.float32)
        z4 = z4 + jnp.dot(p_ref[...], wp_ref[...], preferred_element_type=jnp.float32)
        z4 = z4 + b_ref[...]
        z4_ref[...] = z4
        _stats_accum(st4_ref, z4, pl.program_id(0) == 0)

    return pl.pallas_call(
        body,
        grid=grid,
        in_specs=[
            pl.BlockSpec((NSAMPLE, CQ, 256), lambda i: (0, i, 0)),
            pl.BlockSpec((8, 256), lambda i: (0, 0)),
            pl.BlockSpec((1, 256), lambda i: (0, 0)),
            pl.BlockSpec((1, 256), lambda i: (0, 0)),
            pl.BlockSpec((CQ, C1), lambda i: (i, 0)),
            pl.BlockSpec((256, 256), lambda i: (0, 0)),
            pl.BlockSpec((C1, 256), lambda i: (0, 0)),
            pl.BlockSpec((1, 256), lambda i: (0, 0)),
        ],
        out_specs=[
            pl.BlockSpec((CQ, 256), lambda i: (i, 0)),
            pl.BlockSpec((8, 256), lambda i: (0, 0)),
        ],
        out_shape=[
            jax.ShapeDtypeStruct((NQ, 256), jnp.float32),
            jax.ShapeDtypeStruct((8, 256), jnp.float32),
        ],
    )(z3.reshape(NSAMPLE, NQ, 256), st3, g2, be2, p1, w3m, w3p, b3)


def _layer_out(z4, st4, g3, be3):
    CQ = 4096
    grid = (NQ // CQ,)

    def body(z_ref, st_ref, g_ref, be_ref, o_ref):
        scale, shift = _bn_coeffs(st_ref, g_ref, be_ref, float(NQ))
        o_ref[...] = jnp.maximum(z_ref[...] * scale + shift, 0.0)

    return pl.pallas_call(
        body,
        grid=grid,
        in_specs=[
            pl.BlockSpec((CQ, 256), lambda i: (i, 0)),
            pl.BlockSpec((8, 256), lambda i: (0, 0)),
            pl.BlockSpec((1, 256), lambda i: (0, 0)),
            pl.BlockSpec((1, 256), lambda i: (0, 0)),
        ],
        out_specs=pl.BlockSpec((CQ, 256), lambda i: (i, 0)),
        out_shape=jax.ShapeDtypeStruct((NQ, 256), jnp.float32),
    )(z4, st4, g3, be3)


def kernel(xyz1_proj, xyz2_proj, points1_proj, feat2_proj,
           W0, b0, g0, be0, W1, b1, g1, be1, W2, b2, g2, be2,
           W3, b3, g3, be3):
    f32 = jnp.float32
    t_xyz = xyz2_proj.reshape(TROWS, 3).astype(f32)
    t_feat = feat2_proj.reshape(TROWS, C2).astype(f32)
    table = jnp.zeros((TPAD, GC), f32)
    table = table.at[:TROWS, :3].set(t_xyz).at[:TROWS, 3:3 + C2].set(t_feat)
    x2planes = [jnp.zeros((TPAD,), f32).at[:TROWS].set(t_xyz[:, c])
                for c in range(3)]
    x1flat = xyz1_proj.reshape(NQ, 3).astype(f32)
    x1planes = [x1flat[:, c] for c in range(3)]

    gmat = _sc_select_gather(x2planes, x1planes, table)

    w0cat = jnp.zeros((GC, 128), f32).at[:3 + C2, :].set(W0)
    w0a8 = jnp.zeros((8, 128), f32).at[:3, :].set(W0[:3])
    x1p = jnp.zeros((NQ, 8), f32).at[:, :3].set(x1flat)
    p1 = points1_proj.reshape(NQ, C1).astype(f32)

    z1, st1 = _layer0(gmat, x1p, w0cat, w0a8, b0.reshape(1, 128))
    z2, st2 = _layer_mid(z1, st1, g0.reshape(1, 128), be0.reshape(1, 128),
                         W1, b1.reshape(1, 128), 128)
    z3, st3 = _layer_mid(z2, st2, g1.reshape(1, 128), be1.reshape(1, 128),
                         W2, b2.reshape(1, 256), 256)
    z4, st4 = _layer_pool(z3, st3, g2.reshape(1, 256), be2.reshape(1, 256),
                          p1, W3[:256], W3[256:], b3.reshape(1, 256))
    out = _layer_out(z4, st4, g3.reshape(1, 256), be3.reshape(1, 256))
    return out.reshape(B, H * W, 256)


# speedup vs baseline: 17.1675x; 1.0492x over previous
import functools

import jax
import jax.numpy as jnp
from jax import lax
from jax.experimental import pallas as pl
from jax.experimental.pallas import tpu as pltpu
from jax.experimental.pallas import tpu_sc as plsc

B, H, W, SH, SW = 2, 64, 256, 32, 128
KH, KW = 4, 8
NSAMPLE = 8
DIST = 100.0
C1, C2 = 64, 64
NQ = B * H * W
NTILES = 32
QT = NQ // NTILES
NGROUPS = QT // 16
TROWS = B * SH * SW
ZROW = TROWS
TPAD = 8200
GC = 128
EPS = 1e-5
NROWS = NSAMPLE * NQ


def _min_tree(vals):
    while len(vals) > 1:
        nxt = [jnp.minimum(vals[2 * i], vals[2 * i + 1])
               for i in range(len(vals) // 2)]
        if len(vals) % 2:
            nxt.append(vals[-1])
        vals = nxt
    return vals[0]


def _sc_select_gather(x2planes, x1planes, table):
    mesh = plsc.VectorSubcoreMesh(core_axis_name="c", subcore_axis_name="s")

    @functools.partial(
        pl.kernel,
        out_type=jax.ShapeDtypeStruct((NROWS, GC), jnp.float32),
        mesh=mesh,
        compiler_params=pltpu.CompilerParams(needs_layout_passes=False),
        scratch_types=[
            pltpu.VMEM((TPAD,), jnp.float32),
            pltpu.VMEM((TPAD,), jnp.float32),
            pltpu.VMEM((TPAD,), jnp.float32),
            pltpu.VMEM((QT,), jnp.float32),
            pltpu.VMEM((QT,), jnp.float32),
            pltpu.VMEM((QT,), jnp.float32),
            pltpu.VMEM((NSAMPLE * QT,), jnp.int32),
            pltpu.VMEM((256, GC), jnp.float32),
            pltpu.VMEM((256, GC), jnp.float32),
            pltpu.SemaphoreType.DMA,
            pltpu.SemaphoreType.DMA,
            pltpu.SemaphoreType.DMA,
            pltpu.SemaphoreType.DMA,
        ],
    )
    def body(x2x_h, x2y_h, x2z_h, x1x_h, x1y_h, x1z_h, tab_hbm, out_hbm,
             x2x, x2y, x2z, x1x, x1y, x1z, idxv, gbufa, gbufb,
             gsema, gsemb, osema, osemb):
        cid = lax.axis_index("c")
        sid = lax.axis_index("s")
        wid = sid * 2 + cid
        base = wid * QT
        pltpu.sync_copy(x2x_h, x2x)
        pltpu.sync_copy(x2y_h, x2y)
        pltpu.sync_copy(x2z_h, x2z)
        pltpu.sync_copy(x1x_h.at[pl.ds(base, QT)], x1x)
        pltpu.sync_copy(x1y_h.at[pl.ds(base, QT)], x1y)
        pltpu.sync_copy(x1z_h.at[pl.ds(base, QT)], x1z)

        lanes = lax.iota(jnp.int32, 16)
        sentinel = jnp.int32(0x7FFFFFFF)

        def g_body(g, carry):
            off = g * 16
            q = base + off + lanes
            wq = q & 255
            hq = (q >> 8) & 63
            bq = q >> 14
            h2 = hq >> 1
            w2 = wq >> 1
            x1 = x1x[pl.ds(off, 16)]
            y1 = x1y[pl.ds(off, 16)]
            z1 = x1z[pl.ds(off, 16)]
            keys = []
            for kh in range(KH):
                r = h2 + (kh - KH // 2)
                vr = (r >= 0) & (r < SH)
                rc = jnp.clip(r, 0, SH - 1)
                rbase = (bq << 12) + (rc << 7)
                for kw in range(KW):
                    c = w2 + (kw - KW // 2)
                    v = vr & (c >= 0) & (c < SW)
                    cc = jnp.clip(c, 0, SW - 1)
                    fl = rbase + cc
                    cx = plsc.load_gather(x2x, [fl])
                    cy = plsc.load_gather(x2y, [fl])
                    cz = plsc.load_gather(x2z, [fl])
                    dx = cx - x1
                    dy = cy - y1
                    dz = cz - z1
                    d2 = dx * dx + dy * dy + dz * dz
                    key = (plsc.bitcast(d2, jnp.int32) & jnp.int32(-32)) | (kh * KW + kw)
                    keys.append(jnp.where(v, key, sentinel))
            for s in range(NSAMPLE):
                m = _min_tree(keys)
                ksel = m & 31
                d2s = plsc.bitcast(m & jnp.int32(-32), jnp.float32)
                ok = d2s < jnp.float32(DIST * DIST)
                khs = ksel >> 3
                kws = ksel & 7
                rs = jnp.clip(h2 + khs - KH // 2, 0, SH - 1)
                cs = jnp.clip(w2 + kws - KW // 2, 0, SW - 1)
                fl = (bq << 12) + (rs << 7) + cs
                idxv[pl.ds(s * QT + off, 16)] = jnp.where(ok, fl, jnp.int32(ZROW))
                if s < NSAMPLE - 1:
                    keys = [jnp.where(k == m, sentinel, k) for k in keys]
            return carry

        lax.fori_loop(0, NGROUPS, g_body, 0)

        bufs = (gbufa, gbufb)
        gsems = (gsema, gsemb)
        osems = (osema, osemb)
        gd = [None, None]
        od = [None, None]
        for j in range(33):
            bsel = j & 1
            if j < 32:
                s, part = j >> 2, j & 3
                if od[bsel] is not None:
                    od[bsel].wait()
                    od[bsel] = None
                lo = s * QT + part * 256
                gd[bsel] = (
                    pltpu.async_copy(tab_hbm.at[idxv.at[pl.ds(lo, 128)]],
                                     bufs[bsel].at[pl.ds(0, 128)], gsems[bsel]),
                    pltpu.async_copy(tab_hbm.at[idxv.at[pl.ds(lo + 128, 128)]],
                                     bufs[bsel].at[pl.ds(128, 128)], gsems[bsel]),
                )
            pb = bsel ^ 1
            if gd[pb] is not None:
                gd[pb][0].wait()
                gd[pb][1].wait()
                gd[pb] = None
                jj = j - 1
                s2, part2 = jj >> 2, jj & 3
                od[pb] = pltpu.async_copy(
                    bufs[pb],
                    out_hbm.at[pl.ds(s2 * NQ + base + part2 * 256, 256)],
                    osems[pb])
        for pb in range(2):
            if od[pb] is not None:
                od[pb].wait()

    return body(*x2planes, *x1planes, table)


def _stats_accum(st_ref, z, first):
    @pl.when(first)
    def _():
        st_ref[...] = jnp.zeros_like(st_ref)
    st_ref[0:1, :] += jnp.sum(z, axis=0, keepdims=True)
    st_ref[1:2, :] += jnp.sum(z * z, axis=0, keepdims=True)


def _bn_coeffs(st_ref, g_ref, be_ref, n):
    mu = st_ref[0:1, :] / n
    var = st_ref[1:2, :] / n - mu * mu
    inv = lax.rsqrt(var + EPS)
    scale = inv * g_ref[...]
    shift = be_ref[...] - mu * scale
    return scale, shift


def _z1_block(g_blk, x1_blk, wc_ref, wa_ref, b_ref):
    z = jnp.dot(g_blk, wc_ref[...], preferred_element_type=jnp.float32)
    z = z - jnp.dot(x1_blk, wa_ref[...], preferred_element_type=jnp.float32)
    return z + b_ref[...]


def _layer0_stats(gmat, x1p, w0cat, w0a8, b0):
    CQ = 2048
    grid = (NSAMPLE, NQ // CQ)

    def body(g_ref, x1_ref, wc_ref, wa_ref, b_ref, st_ref):
        z = _z1_block(g_ref[0], x1_ref[...], wc_ref, wa_ref, b_ref)
        i, j = pl.program_id(0), pl.program_id(1)
        _stats_accum(st_ref, z, (i == 0) & (j == 0))

    return pl.pallas_call(
        body,
        grid=grid,
        in_specs=[
            pl.BlockSpec((1, CQ, GC), lambda i, j: (i, j, 0)),
            pl.BlockSpec((CQ, 8), lambda i, j: (j, 0)),
            pl.BlockSpec((GC, 128), lambda i, j: (0, 0)),
            pl.BlockSpec((8, 128), lambda i, j: (0, 0)),
            pl.BlockSpec((1, 128), lambda i, j: (0, 0)),
        ],
        out_specs=pl.BlockSpec((8, 128), lambda i, j: (0, 0)),
        out_shape=jax.ShapeDtypeStruct((8, 128), jnp.float32),
    )(gmat.reshape(NSAMPLE, NQ, GC), x1p, w0cat, w0a8, b0)


def _layer01(gmat, x1p, w0cat, w0a8, b0, st1, g0, be0, W1m, b1):
    CQ = 2048
    grid = (NSAMPLE, NQ // CQ)

    def body(g_ref, x1_ref, wc_ref, wa_ref, b_ref, st_ref, g0_ref, be0_ref,
             w1_ref, b1_ref, z2_ref, st2_ref):
        z = _z1_block(g_ref[0], x1_ref[...], wc_ref, wa_ref, b_ref)
        scale, shift = _bn_coeffs(st_ref, g0_ref, be0_ref, float(NROWS))
        a = jnp.maximum(z * scale + shift, 0.0)
        z2 = jnp.dot(a, w1_ref[...], preferred_element_type=jnp.float32) + b1_ref[...]
        z2_ref[...] = z2
        i, j = pl.program_id(0), pl.program_id(1)
        _stats_accum(st2_ref, z2, (i == 0) & (j == 0))

    return pl.pallas_call(
        body,
        grid=grid,
        in_specs=[
            pl.BlockSpec((1, CQ, GC), lambda i, j: (i, j, 0)),
            pl.BlockSpec((CQ, 8), lambda i, j: (j, 0)),
            pl.BlockSpec((GC, 128), lambda i, j: (0, 0)),
            pl.BlockSpec((8, 128), lambda i, j: (0, 0)),
            pl.BlockSpec((1, 128), lambda i, j: (0, 0)),
            pl.BlockSpec((8, 128), lambda i, j: (0, 0)),
            pl.BlockSpec((1, 128), lambda i, j: (0, 0)),
            pl.BlockSpec((1, 128), lambda i, j: (0, 0)),
            pl.BlockSpec((128, 128), lambda i, j: (0, 0)),
            pl.BlockSpec((1, 128), lambda i, j: (0, 0)),
        ],
        out_specs=[
            pl.BlockSpec((CQ, 128), lambda i, j: (i * (NQ // CQ) + j, 0)),
            pl.BlockSpec((8, 128), lambda i, j: (0, 0)),
        ],
        out_shape=[
            jax.ShapeDtypeStruct((NROWS, 128), jnp.float32),
            jax.ShapeDtypeStruct((8, 128), jnp.float32),
        ],
    )(gmat.reshape(NSAMPLE, NQ, GC), x1p, w0cat, w0a8, b0, st1, g0, be0,
      W1m, b1)


def _layer2_stats(z2, st2, g1, be1, W2m, b2):
    CR = 4096
    grid = (NROWS // CR,)

    def body(z_ref, st_ref, g_ref, be_ref, w_ref, b_ref, st3_ref):
        scale, shift = _bn_coeffs(st_ref, g_ref, be_ref, float(NROWS))
        a = jnp.maximum(z_ref[...] * scale + shift, 0.0)
        z3 = jnp.dot(a, w_ref[...], preferred_element_type=jnp.float32) + b_ref[...]
        _stats_accum(st3_ref, z3, pl.program_id(0) == 0)

    return pl.pallas_call(
        body,
        grid=grid,
        in_specs=[
            pl.BlockSpec((CR, 128), lambda i: (i, 0)),
            pl.BlockSpec((8, 128), lambda i: (0, 0)),
            pl.BlockSpec((1, 128), lambda i: (0, 0)),
            pl.BlockSpec((1, 128), lambda i: (0, 0)),
            pl.BlockSpec((128, 256), lambda i: (0, 0)),
            pl.BlockSpec((1, 256), lambda i: (0, 0)),
        ],
        out_specs=pl.BlockSpec((8, 256), lambda i: (0, 0)),
        out_shape=jax.ShapeDtypeStruct((8, 256), jnp.float32),
    )(z2, st2, g1, be1, W2m, b2)


def _layer23_pool(z2, st2, g1, be1, W2m, b2, st3, g2, be2, p1, w3m, w3p, b3):
    CQ = 1024
    grid = (NQ // CQ,)

    def body(z_ref, st_ref, g_ref, be_ref, w2_ref, b2_ref, st3_ref, g2_ref,
             be2_ref, p_ref, wm_ref, wp_ref, b_ref, z4_ref, st4_ref):
        scale, shift = _bn_coeffs(st_ref, g_ref, be_ref, float(NROWS))
        a = jnp.maximum(z_ref[...] * scale + shift, 0.0)
        a = a.reshape(NSAMPLE * CQ, 128)
        z3 = jnp.dot(a, w2_ref[...], preferred_element_type=jnp.float32) + b2_ref[...]
        scale3, shift3 = _bn_coeffs(st3_ref, g2_ref, be2_ref, float(NROWS))
        a3 = jnp.maximum(z3 * scale3 + shift3, 0.0).reshape(NSAMPLE, CQ, 256)
        m = jnp.max(a3, axis=0)
        z4 = jnp.dot(m, wm_ref[...], preferred_element_type=jnp.float32)
        z4 = z4 + jnp.dot(p_ref[...], wp_ref[...], preferred_element_type=jnp.float32)
        z4 = z4 + b_ref[...]
        z4_ref[...] = z4
        _stats_accum(st4_ref, z4, pl.program_id(0) == 0)

    return pl.pallas_call(
        body,
        grid=grid,
        in_specs=[
            pl.BlockSpec((NSAMPLE, CQ, 128), lambda i: (0, i, 0)),
            pl.BlockSpec((8, 128), lambda i: (0, 0)),
            pl.BlockSpec((1, 128), lambda i: (0, 0)),
            pl.BlockSpec((1, 128), lambda i: (0, 0)),
            pl.BlockSpec((128, 256), lambda i: (0, 0)),
            pl.BlockSpec((1, 256), lambda i: (0, 0)),
            pl.BlockSpec((8, 256), lambda i: (0, 0)),
            pl.BlockSpec((1, 256), lambda i: (0, 0)),
            pl.BlockSpec((1, 256), lambda i: (0, 0)),
            pl.BlockSpec((CQ, C1), lambda i: (i, 0)),
            pl.BlockSpec((256, 256), lambda i: (0, 0)),
            pl.BlockSpec((C1, 256), lambda i: (0, 0)),
            pl.BlockSpec((1, 256), lambda i: (0, 0)),
        ],
        out_specs=[
            pl.BlockSpec((CQ, 256), lambda i: (i, 0)),
            pl.BlockSpec((8, 256), lambda i: (0, 0)),
        ],
        out_shape=[
            jax.ShapeDtypeStruct((NQ, 256), jnp.float32),
            jax.ShapeDtypeStruct((8, 256), jnp.float32),
        ],
    )(z2.reshape(NSAMPLE, NQ, 128), st2, g1, be1, W2m, b2, st3, g2, be2,
      p1, w3m, w3p, b3)


def _layer_out(z4, st4, g3, be3):
    CQ = 4096
    grid = (NQ // CQ,)

    def body(z_ref, st_ref, g_ref, be_ref, o_ref):
        scale, shift = _bn_coeffs(st_ref, g_ref, be_ref, float(NQ))
        o_ref[...] = jnp.maximum(z_ref[...] * scale + shift, 0.0)

    return pl.pallas_call(
        body,
        grid=grid,
        in_specs=[
            pl.BlockSpec((CQ, 256), lambda i: (i, 0)),
            pl.BlockSpec((8, 256), lambda i: (0, 0)),
            pl.BlockSpec((1, 256), lambda i: (0, 0)),
            pl.BlockSpec((1, 256), lambda i: (0, 0)),
        ],
        out_specs=pl.BlockSpec((CQ, 256), lambda i: (i, 0)),
        out_shape=jax.ShapeDtypeStruct((NQ, 256), jnp.float32),
    )(z4, st4, g3, be3)


def kernel(xyz1_proj, xyz2_proj, points1_proj, feat2_proj,
           W0, b0, g0, be0, W1, b1, g1, be1, W2, b2, g2, be2,
           W3, b3, g3, be3):
    f32 = jnp.float32
    t_xyz = xyz2_proj.reshape(TROWS, 3).astype(f32)
    t_feat = feat2_proj.reshape(TROWS, C2).astype(f32)
    table = jnp.zeros((TPAD, GC), f32)
    table = table.at[:TROWS, :3].set(t_xyz).at[:TROWS, 3:3 + C2].set(t_feat)
    x2planes = [jnp.zeros((TPAD,), f32).at[:TROWS].set(t_xyz[:, c])
                for c in range(3)]
    x1flat = xyz1_proj.reshape(NQ, 3).astype(f32)
    x1planes = [x1flat[:, c] for c in range(3)]

    gmat = _sc_select_gather(x2planes, x1planes, table)

    w0cat = jnp.zeros((GC, 128), f32).at[:3 + C2, :].set(W0)
    w0a8 = jnp.zeros((8, 128), f32).at[:3, :].set(W0[:3])
    x1p = jnp.zeros((NQ, 8), f32).at[:, :3].set(x1flat)
    p1 = points1_proj.reshape(NQ, C1).astype(f32)

    b0r, b1r = b0.reshape(1, 128), b1.reshape(1, 128)
    b2r, b3r = b2.reshape(1, 256), b3.reshape(1, 256)
    st1 = _layer0_stats(gmat, x1p, w0cat, w0a8, b0r)
    z2, st2 = _layer01(gmat, x1p, w0cat, w0a8, b0r, st1,
                       g0.reshape(1, 128), be0.reshape(1, 128), W1, b1r)
    st3 = _layer2_stats(z2, st2, g1.reshape(1, 128), be1.reshape(1, 128),
                        W2, b2r)
    z4, st4 = _layer23_pool(z2, st2, g1.reshape(1, 128), be1.reshape(1, 128),
                            W2, b2r, st3, g2.reshape(1, 256),
                            be2.reshape(1, 256), p1, W3[:256], W3[256:], b3r)
    out = _layer_out(z4, st4, g3.reshape(1, 256), be3.reshape(1, 256))
    return out.reshape(B, H * W, 256)


# speedup vs baseline: 17.2992x; 1.0077x over previous
import functools

import jax
import jax.numpy as jnp
from jax import lax
from jax.experimental import pallas as pl
from jax.experimental.pallas import tpu as pltpu
from jax.experimental.pallas import tpu_sc as plsc

B, H, W, SH, SW = 2, 64, 256, 32, 128
KH, KW = 4, 8
NSAMPLE = 8
DIST = 100.0
C1, C2 = 64, 64
NQ = B * H * W
NTILES = 32
QT = NQ // NTILES
NGROUPS = QT // 16
TROWS = B * SH * SW
ZROW = TROWS
TPAD = 8200
GC = 128
EPS = 1e-5
NROWS = NSAMPLE * NQ


def _min_tree(vals):
    while len(vals) > 1:
        nxt = [jnp.minimum(vals[2 * i], vals[2 * i + 1])
               for i in range(len(vals) // 2)]
        if len(vals) % 2:
            nxt.append(vals[-1])
        vals = nxt
    return vals[0]


def _sc_select_gather(x2planes, x1planes, table):
    mesh = plsc.VectorSubcoreMesh(core_axis_name="c", subcore_axis_name="s")

    @functools.partial(
        pl.kernel,
        out_type=jax.ShapeDtypeStruct((NROWS, GC), jnp.float32),
        mesh=mesh,
        compiler_params=pltpu.CompilerParams(needs_layout_passes=False),
        scratch_types=[
            pltpu.VMEM((TPAD,), jnp.float32),
            pltpu.VMEM((TPAD,), jnp.float32),
            pltpu.VMEM((TPAD,), jnp.float32),
            pltpu.VMEM((QT,), jnp.float32),
            pltpu.VMEM((QT,), jnp.float32),
            pltpu.VMEM((QT,), jnp.float32),
            pltpu.VMEM((NSAMPLE * QT,), jnp.int32),
            pltpu.VMEM((256, GC), jnp.float32),
            pltpu.VMEM((256, GC), jnp.float32),
            pltpu.SemaphoreType.DMA,
            pltpu.SemaphoreType.DMA,
            pltpu.SemaphoreType.DMA,
            pltpu.SemaphoreType.DMA,
        ],
    )
    def body(x2x_h, x2y_h, x2z_h, x1x_h, x1y_h, x1z_h, tab_hbm, out_hbm,
             x2x, x2y, x2z, x1x, x1y, x1z, idxv, gbufa, gbufb,
             gsema, gsemb, osema, osemb):
        cid = lax.axis_index("c")
        sid = lax.axis_index("s")
        wid = sid * 2 + cid
        base = wid * QT
        pltpu.sync_copy(x2x_h, x2x)
        pltpu.sync_copy(x2y_h, x2y)
        pltpu.sync_copy(x2z_h, x2z)
        pltpu.sync_copy(x1x_h.at[pl.ds(base, QT)], x1x)
        pltpu.sync_copy(x1y_h.at[pl.ds(base, QT)], x1y)
        pltpu.sync_copy(x1z_h.at[pl.ds(base, QT)], x1z)

        lanes = lax.iota(jnp.int32, 16)
        sentinel = jnp.int32(0x7FFFFFFF)

        def g_body(g, carry):
            off = g * 16
            q = base + off + lanes
            wq = q & 255
            hq = (q >> 8) & 63
            bq = q >> 14
            h2 = hq >> 1
            w2 = wq >> 1
            x1 = x1x[pl.ds(off, 16)]
            y1 = x1y[pl.ds(off, 16)]
            z1 = x1z[pl.ds(off, 16)]
            keys = []
            for kh in range(KH):
                r = h2 + (kh - KH // 2)
                vr = (r >= 0) & (r < SH)
                rc = jnp.clip(r, 0, SH - 1)
                rbase = (bq << 12) + (rc << 7)
                for kw in range(KW):
                    c = w2 + (kw - KW // 2)
                    v = vr & (c >= 0) & (c < SW)
                    cc = jnp.clip(c, 0, SW - 1)
                    fl = rbase + cc
                    cx = plsc.load_gather(x2x, [fl])
                    cy = plsc.load_gather(x2y, [fl])
                    cz = plsc.load_gather(x2z, [fl])
                    dx = cx - x1
                    dy = cy - y1
                    dz = cz - z1
                    d2 = dx * dx + dy * dy + dz * dz
                    key = (plsc.bitcast(d2, jnp.int32) & jnp.int32(-32)) | (kh * KW + kw)
                    keys.append(jnp.where(v, key, sentinel))
            for s in range(NSAMPLE):
                m = _min_tree(keys)
                ksel = m & 31
                d2s = plsc.bitcast(m & jnp.int32(-32), jnp.float32)
                ok = d2s < jnp.float32(DIST * DIST)
                khs = ksel >> 3
                kws = ksel & 7
                rs = jnp.clip(h2 + khs - KH // 2, 0, SH - 1)
                cs = jnp.clip(w2 + kws - KW // 2, 0, SW - 1)
                fl = (bq << 12) + (rs << 7) + cs
                idxv[pl.ds(s * QT + off, 16)] = jnp.where(ok, fl, jnp.int32(ZROW))
                if s < NSAMPLE - 1:
                    keys = [jnp.where(k == m, sentinel, k) for k in keys]
            return carry

        lax.fori_loop(0, NGROUPS, g_body, 0)

        bufs = (gbufa, gbufb)
        gsems = (gsema, gsemb)
        osems = (osema, osemb)
        gd = [None, None]
        od = [None, None]
        for j in range(33):
            bsel = j & 1
            if j < 32:
                s, part = j >> 2, j & 3
                if od[bsel] is not None:
                    od[bsel].wait()
                    od[bsel] = None
                lo = s * QT + part * 256
                gd[bsel] = (
                    pltpu.async_copy(tab_hbm.at[idxv.at[pl.ds(lo, 128)]],
                                     bufs[bsel].at[pl.ds(0, 128)], gsems[bsel]),
                    pltpu.async_copy(tab_hbm.at[idxv.at[pl.ds(lo + 128, 128)]],
                                     bufs[bsel].at[pl.ds(128, 128)], gsems[bsel]),
                )
            pb = bsel ^ 1
            if gd[pb] is not None:
                gd[pb][0].wait()
                gd[pb][1].wait()
                gd[pb] = None
                jj = j - 1
                s2, part2 = jj >> 2, jj & 3
                od[pb] = pltpu.async_copy(
                    bufs[pb],
                    out_hbm.at[pl.ds(s2 * NQ + base + part2 * 256, 256)],
                    osems[pb])
        for pb in range(2):
            if od[pb] is not None:
                od[pb].wait()

    return body(*x2planes, *x1planes, table)


def _stats_accum(st_ref, z, first):
    @pl.when(first)
    def _():
        st_ref[...] = jnp.zeros_like(st_ref)
    st_ref[0:1, :] += jnp.sum(z, axis=0, keepdims=True)
    st_ref[1:2, :] += jnp.sum(z * z, axis=0, keepdims=True)


def _bn_coeffs(st_ref, g_ref, be_ref, n):
    mu = st_ref[0:1, :] / n
    var = st_ref[1:2, :] / n - mu * mu
    inv = lax.rsqrt(var + EPS)
    scale = inv * g_ref[...]
    shift = be_ref[...] - mu * scale
    return scale, shift


def _z1_block(g_blk, x1_blk, wc_ref, wa_ref, b_ref):
    z = jnp.dot(g_blk, wc_ref[...], preferred_element_type=jnp.float32)
    z = z - jnp.dot(x1_blk, wa_ref[...], preferred_element_type=jnp.float32)
    return z + b_ref[...]


def _layer01(gmat, x1p, w0cat, w0a8, b0, g0, be0, W1m, b1):
    CQ = 2048
    NJ = NQ // CQ
    grid = (2, NSAMPLE, NJ)

    def body(g_ref, x1_ref, wc_ref, wa_ref, b_ref, g0_ref, be0_ref,
             w1_ref, b1_ref, z2_ref, st2_ref, st1v):
        p, i, j = pl.program_id(0), pl.program_id(1), pl.program_id(2)
        z = _z1_block(g_ref[0], x1_ref[...], wc_ref, wa_ref, b_ref)

        @pl.when(p == 0)
        def _():
            _stats_accum(st1v, z, (i == 0) & (j == 0))

        @pl.when(p == 1)
        def _():
            scale, shift = _bn_coeffs(st1v, g0_ref, be0_ref, float(NROWS))
            a = jnp.maximum(z * scale + shift, 0.0)
            z2 = jnp.dot(a, w1_ref[...], preferred_element_type=jnp.float32)
            z2_ref[0] = z2 + b1_ref[...]
            _stats_accum(st2_ref, z2_ref[0], (i == 0) & (j == 0))

    return pl.pallas_call(
        body,
        grid=grid,
        in_specs=[
            pl.BlockSpec((1, CQ, GC), lambda p, i, j: (i, j, 0)),
            pl.BlockSpec((CQ, 8), lambda p, i, j: (j, 0)),
            pl.BlockSpec((GC, 128), lambda p, i, j: (0, 0)),
            pl.BlockSpec((8, 128), lambda p, i, j: (0, 0)),
            pl.BlockSpec((1, 128), lambda p, i, j: (0, 0)),
            pl.BlockSpec((1, 128), lambda p, i, j: (0, 0)),
            pl.BlockSpec((1, 128), lambda p, i, j: (0, 0)),
            pl.BlockSpec((128, 128), lambda p, i, j: (0, 0)),
            pl.BlockSpec((1, 128), lambda p, i, j: (0, 0)),
        ],
        out_specs=[
            pl.BlockSpec((1, CQ, 128),
                         lambda p, i, j: (jnp.where(p == 0, NSAMPLE, i),
                                          jnp.where(p == 0, 0, j), 0)),
            pl.BlockSpec((8, 128), lambda p, i, j: (0, 0)),
        ],
        out_shape=[
            jax.ShapeDtypeStruct((NSAMPLE + 1, NQ, 128), jnp.float32),
            jax.ShapeDtypeStruct((8, 128), jnp.float32),
        ],
        scratch_shapes=[pltpu.VMEM((8, 128), jnp.float32)],
    )(gmat.reshape(NSAMPLE, NQ, GC), x1p, w0cat, w0a8, b0, g0, be0, W1m, b1)


def _layer23_pool(z2, st2, g1, be1, W2m, b2, g2, be2, p1, w3m, w3p, b3):
    CQ = 1024
    NJ = NQ // CQ
    grid = (2, NJ)

    def body(z_ref, st_ref, g_ref, be_ref, w2_ref, b2_ref, g2_ref,
             be2_ref, p_ref, wm_ref, wp_ref, b_ref, z4_ref, st4_ref, st3v):
        p, i = pl.program_id(0), pl.program_id(1)
        scale, shift = _bn_coeffs(st_ref, g_ref, be_ref, float(NROWS))
        a = jnp.maximum(z_ref[...] * scale + shift, 0.0)
        a = a.reshape(NSAMPLE * CQ, 128)
        z3 = jnp.dot(a, w2_ref[...], preferred_element_type=jnp.float32) + b2_ref[...]

        @pl.when(p == 0)
        def _():
            _stats_accum(st3v, z3, i == 0)

        @pl.when(p == 1)
        def _():
            scale3, shift3 = _bn_coeffs(st3v, g2_ref, be2_ref, float(NROWS))
            a3 = jnp.maximum(z3 * scale3 + shift3, 0.0).reshape(NSAMPLE, CQ, 256)
            m = jnp.max(a3, axis=0)
            z4 = jnp.dot(m, wm_ref[...], preferred_element_type=jnp.float32)
            z4 = z4 + jnp.dot(p_ref[...], wp_ref[...],
                              preferred_element_type=jnp.float32)
            z4_ref[...] = z4 + b_ref[...]
            _stats_accum(st4_ref, z4_ref[...], i == 0)

    return pl.pallas_call(
        body,
        grid=grid,
        in_specs=[
            pl.BlockSpec((NSAMPLE, CQ, 128), lambda p, i: (0, i, 0)),
            pl.BlockSpec((8, 128), lambda p, i: (0, 0)),
            pl.BlockSpec((1, 128), lambda p, i: (0, 0)),
            pl.BlockSpec((1, 128), lambda p, i: (0, 0)),
            pl.BlockSpec((128, 256), lambda p, i: (0, 0)),
            pl.BlockSpec((1, 256), lambda p, i: (0, 0)),
            pl.BlockSpec((1, 256), lambda p, i: (0, 0)),
            pl.BlockSpec((1, 256), lambda p, i: (0, 0)),
            pl.BlockSpec((CQ, C1), lambda p, i: (i, 0)),
            pl.BlockSpec((256, 256), lambda p, i: (0, 0)),
            pl.BlockSpec((C1, 256), lambda p, i: (0, 0)),
            pl.BlockSpec((1, 256), lambda p, i: (0, 0)),
        ],
        out_specs=[
            pl.BlockSpec((CQ, 256),
                         lambda p, i: (jnp.where(p == 0, NJ, i), 0)),
            pl.BlockSpec((8, 256), lambda p, i: (0, 0)),
        ],
        out_shape=[
            jax.ShapeDtypeStruct((NQ + CQ, 256), jnp.float32),
            jax.ShapeDtypeStruct((8, 256), jnp.float32),
        ],
        scratch_shapes=[pltpu.VMEM((8, 256), jnp.float32)],
    )(z2, st2, g1, be1, W2m, b2, g2, be2, p1, w3m, w3p, b3)


def _layer_out(z4, st4, g3, be3):
    CQ = 4096
    grid = (NQ // CQ,)

    def body(z_ref, st_ref, g_ref, be_ref, o_ref):
        scale, shift = _bn_coeffs(st_ref, g_ref, be_ref, float(NQ))
        o_ref[...] = jnp.maximum(z_ref[...] * scale + shift, 0.0)

    return pl.pallas_call(
        body,
        grid=grid,
        in_specs=[
            pl.BlockSpec((CQ, 256), lambda i: (i, 0)),
            pl.BlockSpec((8, 256), lambda i: (0, 0)),
            pl.BlockSpec((1, 256), lambda i: (0, 0)),
            pl.BlockSpec((1, 256), lambda i: (0, 0)),
        ],
        out_specs=pl.BlockSpec((CQ, 256), lambda i: (i, 0)),
        out_shape=jax.ShapeDtypeStruct((NQ, 256), jnp.float32),
    )(z4, st4, g3, be3)


def kernel(xyz1_proj, xyz2_proj, points1_proj, feat2_proj,
           W0, b0, g0, be0, W1, b1, g1, be1, W2, b2, g2, be2,
           W3, b3, g3, be3):
    f32 = jnp.float32
    t_xyz = xyz2_proj.reshape(TROWS, 3).astype(f32)
    t_feat = feat2_proj.reshape(TROWS, C2).astype(f32)
    table = jnp.zeros((TPAD, GC), f32)
    table = table.at[:TROWS, :3].set(t_xyz).at[:TROWS, 3:3 + C2].set(t_feat)
    x2planes = [jnp.zeros((TPAD,), f32).at[:TROWS].set(t_xyz[:, c])
                for c in range(3)]
    x1flat = xyz1_proj.reshape(NQ, 3).astype(f32)
    x1planes = [x1flat[:, c] for c in range(3)]

    gmat = _sc_select_gather(x2planes, x1planes, table)

    w0cat = jnp.zeros((GC, 128), f32).at[:3 + C2, :].set(W0)
    w0a8 = jnp.zeros((8, 128), f32).at[:3, :].set(W0[:3])
    x1p = jnp.zeros((NQ, 8), f32).at[:, :3].set(x1flat)
    p1 = points1_proj.reshape(NQ, C1).astype(f32)

    b0r, b1r = b0.reshape(1, 128), b1.reshape(1, 128)
    b2r, b3r = b2.reshape(1, 256), b3.reshape(1, 256)
    z2, st2 = _layer01(gmat, x1p, w0cat, w0a8, b0r,
                       g0.reshape(1, 128), be0.reshape(1, 128), W1, b1r)
    z4, st4 = _layer23_pool(z2, st2, g1.reshape(1, 128), be1.reshape(1, 128),
                            W2, b2r, g2.reshape(1, 256),
                            be2.reshape(1, 256), p1, W3[:256], W3[256:], b3r)
    out = _layer_out(z4, st4, g3.reshape(1, 256), be3.reshape(1, 256))
    return out.reshape(B, H * W, 256)


# speedup vs baseline: 17.6886x; 1.0225x over previous
import functools

import jax
import jax.numpy as jnp
from jax import lax
from jax.experimental import pallas as pl
from jax.experimental.pallas import tpu as pltpu
from jax.experimental.pallas import tpu_sc as plsc

B, H, W, SH, SW = 2, 64, 256, 32, 128
KH, KW = 4, 8
NSAMPLE = 8
DIST = 100.0
C1, C2 = 64, 64
NQ = B * H * W
NTILES = 32
QT = NQ // NTILES
NGROUPS = QT // 16
TROWS = B * SH * SW
ZROW = TROWS
TPAD = 8200
GC = 128
EPS = 1e-5
NROWS = NSAMPLE * NQ


def _min_tree_kv(keys, idxs):
    keys = list(keys)
    idxs = list(idxs)
    while len(keys) > 1:
        nk, ni = [], []
        for i in range(len(keys) // 2):
            le = keys[2 * i] <= keys[2 * i + 1]
            nk.append(jnp.where(le, keys[2 * i], keys[2 * i + 1]))
            ni.append(jnp.where(le, idxs[2 * i], idxs[2 * i + 1]))
        if len(keys) % 2:
            nk.append(keys[-1])
            ni.append(idxs[-1])
        keys, idxs = nk, ni
    return keys[0], idxs[0]


def _sc_select_gather(x2planes, x1planes, table):
    mesh = plsc.VectorSubcoreMesh(core_axis_name="c", subcore_axis_name="s")

    @functools.partial(
        pl.kernel,
        out_type=jax.ShapeDtypeStruct((NROWS, GC), jnp.float32),
        mesh=mesh,
        compiler_params=pltpu.CompilerParams(needs_layout_passes=False),
        scratch_types=[
            pltpu.VMEM((TPAD,), jnp.float32),
            pltpu.VMEM((TPAD,), jnp.float32),
            pltpu.VMEM((TPAD,), jnp.float32),
            pltpu.VMEM((QT,), jnp.float32),
            pltpu.VMEM((QT,), jnp.float32),
            pltpu.VMEM((QT,), jnp.float32),
            pltpu.VMEM((NSAMPLE * QT,), jnp.int32),
            pltpu.VMEM((256, GC), jnp.float32),
            pltpu.VMEM((256, GC), jnp.float32),
            pltpu.SemaphoreType.DMA,
            pltpu.SemaphoreType.DMA,
            pltpu.SemaphoreType.DMA,
            pltpu.SemaphoreType.DMA,
        ],
    )
    def body(x2x_h, x2y_h, x2z_h, x1x_h, x1y_h, x1z_h, tab_hbm, out_hbm,
             x2x, x2y, x2z, x1x, x1y, x1z, idxv, gbufa, gbufb,
             gsema, gsemb, osema, osemb):
        cid = lax.axis_index("c")
        sid = lax.axis_index("s")
        wid = sid * 2 + cid
        base = wid * QT
        pltpu.sync_copy(x2x_h, x2x)
        pltpu.sync_copy(x2y_h, x2y)
        pltpu.sync_copy(x2z_h, x2z)
        pltpu.sync_copy(x1x_h.at[pl.ds(base, QT)], x1x)
        pltpu.sync_copy(x1y_h.at[pl.ds(base, QT)], x1y)
        pltpu.sync_copy(x1z_h.at[pl.ds(base, QT)], x1z)

        lanes = lax.iota(jnp.int32, 16)
        sentinel = jnp.int32(0x7FFFFFFF)

        def g_body(g, carry):
            off = g * 16
            q = base + off + lanes
            wq = q & 255
            hq = (q >> 8) & 63
            bq = q >> 14
            h2 = hq >> 1
            w2 = wq >> 1
            x1 = x1x[pl.ds(off, 16)]
            y1 = x1y[pl.ds(off, 16)]
            z1 = x1z[pl.ds(off, 16)]
            zero16 = lanes - lanes
            keys = []
            kids = []
            for kh in range(KH):
                r = h2 + (kh - KH // 2)
                vr = (r >= 0) & (r < SH)
                rc = jnp.clip(r, 0, SH - 1)
                rbase = (bq << 12) + (rc << 7)
                for kw in range(KW):
                    c = w2 + (kw - KW // 2)
                    v = vr & (c >= 0) & (c < SW)
                    cc = jnp.clip(c, 0, SW - 1)
                    fl = rbase + cc
                    cx = plsc.load_gather(x2x, [fl])
                    cy = plsc.load_gather(x2y, [fl])
                    cz = plsc.load_gather(x2z, [fl])
                    dx = cx - x1
                    dy = cy - y1
                    dz = cz - z1
                    d2 = dx * dx + dy * dy + dz * dz
                    keys.append(jnp.where(v, plsc.bitcast(d2, jnp.int32),
                                          sentinel))
                    kids.append(zero16 + (kh * KW + kw))
            for s in range(NSAMPLE):
                m, ksel = _min_tree_kv(keys, kids)
                d2s = plsc.bitcast(m, jnp.float32)
                ok = d2s < jnp.float32(DIST * DIST)
                khs = ksel >> 3
                kws = ksel & 7
                rs = jnp.clip(h2 + khs - KH // 2, 0, SH - 1)
                cs = jnp.clip(w2 + kws - KW // 2, 0, SW - 1)
                fl = (bq << 12) + (rs << 7) + cs
                idxv[pl.ds(s * QT + off, 16)] = jnp.where(ok, fl, jnp.int32(ZROW))
                if s < NSAMPLE - 1:
                    keys = [jnp.where(ksel == kk, sentinel, k)
                            for k, kk in zip(keys, range(32))]
            return carry

        lax.fori_loop(0, NGROUPS, g_body, 0)

        bufs = (gbufa, gbufb)
        gsems = (gsema, gsemb)
        osems = (osema, osemb)
        gd = [None, None]
        od = [None, None]
        for j in range(33):
            bsel = j & 1
            if j < 32:
                s, part = j >> 2, j & 3
                if od[bsel] is not None:
                    od[bsel].wait()
                    od[bsel] = None
                lo = s * QT + part * 256
                gd[bsel] = (
                    pltpu.async_copy(tab_hbm.at[idxv.at[pl.ds(lo, 128)]],
                                     bufs[bsel].at[pl.ds(0, 128)], gsems[bsel]),
                    pltpu.async_copy(tab_hbm.at[idxv.at[pl.ds(lo + 128, 128)]],
                                     bufs[bsel].at[pl.ds(128, 128)], gsems[bsel]),
                )
            pb = bsel ^ 1
            if gd[pb] is not None:
                gd[pb][0].wait()
                gd[pb][1].wait()
                gd[pb] = None
                jj = j - 1
                s2, part2 = jj >> 2, jj & 3
                od[pb] = pltpu.async_copy(
                    bufs[pb],
                    out_hbm.at[pl.ds(s2 * NQ + base + part2 * 256, 256)],
                    osems[pb])
        for pb in range(2):
            if od[pb] is not None:
                od[pb].wait()

    return body(*x2planes, *x1planes, table)


def _stats_accum(st_ref, z, first):
    @pl.when(first)
    def _():
        st_ref[...] = jnp.zeros_like(st_ref)
    st_ref[0:1, :] += jnp.sum(z, axis=0, keepdims=True)
    st_ref[1:2, :] += jnp.sum(z * z, axis=0, keepdims=True)


def _bn_coeffs(st_ref, g_ref, be_ref, n):
    mu = st_ref[0:1, :] / n
    var = st_ref[1:2, :] / n - mu * mu
    inv = lax.rsqrt(var + EPS)
    scale = inv * g_ref[...]
    shift = be_ref[...] - mu * scale
    return scale, shift


def _z1_block(g_blk, x1_blk, wc_ref, wa_ref, b_ref):
    z = jnp.dot(g_blk, wc_ref[...], preferred_element_type=jnp.float32)
    z = z - jnp.dot(x1_blk, wa_ref[...], preferred_element_type=jnp.float32)
    return z + b_ref[...]


def _layer01(gmat, x1p, w0cat, w0a8, b0, g0, be0, W1m, b1):
    CQ = 2048
    NJ = NQ // CQ
    grid = (2, NSAMPLE, NJ)

    def body(g_ref, x1_ref, wc_ref, wa_ref, b_ref, g0_ref, be0_ref,
             w1_ref, b1_ref, z2_ref, st2_ref, st1v):
        p, i, j = pl.program_id(0), pl.program_id(1), pl.program_id(2)
        z = _z1_block(g_ref[0], x1_ref[...], wc_ref, wa_ref, b_ref)

        @pl.when(p == 0)
        def _():
            _stats_accum(st1v, z, (i == 0) & (j == 0))

        @pl.when(p == 1)
        def _():
            scale, shift = _bn_coeffs(st1v, g0_ref, be0_ref, float(NROWS))
            a = jnp.maximum(z * scale + shift, 0.0)
            z2 = jnp.dot(a, w1_ref[...], preferred_element_type=jnp.float32)
            z2 = z2 + b1_ref[...]
            z2_ref[0] = z2.astype(jnp.bfloat16)
            _stats_accum(st2_ref, z2, (i == 0) & (j == 0))

    return pl.pallas_call(
        body,
        grid=grid,
        in_specs=[
            pl.BlockSpec((1, CQ, GC), lambda p, i, j: (i, j, 0)),
            pl.BlockSpec((CQ, 8), lambda p, i, j: (j, 0)),
            pl.BlockSpec((GC, 128), lambda p, i, j: (0, 0)),
            pl.BlockSpec((8, 128), lambda p, i, j: (0, 0)),
            pl.BlockSpec((1, 128), lambda p, i, j: (0, 0)),
            pl.BlockSpec((1, 128), lambda p, i, j: (0, 0)),
            pl.BlockSpec((1, 128), lambda p, i, j: (0, 0)),
            pl.BlockSpec((128, 128), lambda p, i, j: (0, 0)),
            pl.BlockSpec((1, 128), lambda p, i, j: (0, 0)),
        ],
        out_specs=[
            pl.BlockSpec((1, CQ, 128),
                         lambda p, i, j: (jnp.where(p == 0, NSAMPLE, i),
                                          jnp.where(p == 0, 0, j), 0)),
            pl.BlockSpec((8, 128), lambda p, i, j: (0, 0)),
        ],
        out_shape=[
            jax.ShapeDtypeStruct((NSAMPLE + 1, NQ, 128), jnp.bfloat16),
            jax.ShapeDtypeStruct((8, 128), jnp.float32),
        ],
        scratch_shapes=[pltpu.VMEM((8, 128), jnp.float32)],
    )(gmat.reshape(NSAMPLE, NQ, GC), x1p, w0cat, w0a8, b0, g0, be0, W1m, b1)


def _layer234(z2, st2, g1, be1, W2m, b2, g2, be2, p1, w3m, w3p, b3, g3, be3):
    CQ = 1024
    NJ = NQ // CQ
    grid = (3, NJ)

    def body(z_ref, st_ref, g_ref, be_ref, w2_ref, b2_ref, g2_ref,
             be2_ref, p_ref, wm_ref, wp_ref, b_ref, g3_ref, be3_ref,
             o_ref, st3v, st4v, z4v):
        p, i = pl.program_id(0), pl.program_id(1)

        @pl.when(p < 2)
        def _():
            scale, shift = _bn_coeffs(st_ref, g_ref, be_ref, float(NROWS))
            a = jnp.maximum(z_ref[...].astype(jnp.float32) * scale + shift, 0.0)
            a = a.reshape(NSAMPLE * CQ, 128)
            z3 = jnp.dot(a, w2_ref[...],
                         preferred_element_type=jnp.float32) + b2_ref[...]

            @pl.when(p == 0)
            def _():
                _stats_accum(st3v, z3, i == 0)

            @pl.when(p == 1)
            def _():
                scale3, shift3 = _bn_coeffs(st3v, g2_ref, be2_ref, float(NROWS))
                a3 = jnp.maximum(z3 * scale3 + shift3,
                                 0.0).reshape(NSAMPLE, CQ, 256)
                m = jnp.max(a3, axis=0)
                z4 = jnp.dot(m, wm_ref[...], preferred_element_type=jnp.float32)
                z4 = z4 + jnp.dot(p_ref[...], wp_ref[...],
                                  preferred_element_type=jnp.float32)
                z4 = z4 + b_ref[...]
                z4v[i] = z4
                _stats_accum(st4v, z4, i == 0)

        @pl.when(p == 2)
        def _():
            scale4, shift4 = _bn_coeffs(st4v, g3_ref, be3_ref, float(NQ))
            o_ref[...] = jnp.maximum(z4v[i] * scale4 + shift4, 0.0)

    return pl.pallas_call(
        body,
        grid=grid,
        in_specs=[
            pl.BlockSpec((NSAMPLE, CQ, 128),
                         lambda p, i: (0, jnp.where(p == 2, 0, i), 0)),
            pl.BlockSpec((8, 128), lambda p, i: (0, 0)),
            pl.BlockSpec((1, 128), lambda p, i: (0, 0)),
            pl.BlockSpec((1, 128), lambda p, i: (0, 0)),
            pl.BlockSpec((128, 256), lambda p, i: (0, 0)),
            pl.BlockSpec((1, 256), lambda p, i: (0, 0)),
            pl.BlockSpec((1, 256), lambda p, i: (0, 0)),
            pl.BlockSpec((1, 256), lambda p, i: (0, 0)),
            pl.BlockSpec((CQ, C1), lambda p, i: (jnp.where(p == 2, 0, i), 0)),
            pl.BlockSpec((256, 256), lambda p, i: (0, 0)),
            pl.BlockSpec((C1, 256), lambda p, i: (0, 0)),
            pl.BlockSpec((1, 256), lambda p, i: (0, 0)),
            pl.BlockSpec((1, 256), lambda p, i: (0, 0)),
            pl.BlockSpec((1, 256), lambda p, i: (0, 0)),
        ],
        out_specs=pl.BlockSpec((CQ, 256),
                               lambda p, i: (jnp.where(p == 2, i, 0), 0)),
        out_shape=jax.ShapeDtypeStruct((NQ, 256), jnp.float32),
        scratch_shapes=[
            pltpu.VMEM((8, 256), jnp.float32),
            pltpu.VMEM((8, 256), jnp.float32),
            pltpu.VMEM((NJ, CQ, 256), jnp.float32),
        ],
    )(z2, st2, g1, be1, W2m, b2, g2, be2, p1, w3m, w3p, b3, g3, be3)


def kernel(xyz1_proj, xyz2_proj, points1_proj, feat2_proj,
           W0, b0, g0, be0, W1, b1, g1, be1, W2, b2, g2, be2,
           W3, b3, g3, be3):
    f32 = jnp.float32
    t_xyz = xyz2_proj.reshape(TROWS, 3).astype(f32)
    t_feat = feat2_proj.reshape(TROWS, C2).astype(f32)
    table = jnp.zeros((TPAD, GC), f32)
    table = table.at[:TROWS, :3].set(t_xyz).at[:TROWS, 3:3 + C2].set(t_feat)
    x2planes = [jnp.zeros((TPAD,), f32).at[:TROWS].set(t_xyz[:, c])
                for c in range(3)]
    x1flat = xyz1_proj.reshape(NQ, 3).astype(f32)
    x1planes = [x1flat[:, c] for c in range(3)]

    gmat = _sc_select_gather(x2planes, x1planes, table)

    w0cat = jnp.zeros((GC, 128), f32).at[:3 + C2, :].set(W0)
    w0a8 = jnp.zeros((8, 128), f32).at[:3, :].set(W0[:3])
    x1p = jnp.zeros((NQ, 8), f32).at[:, :3].set(x1flat)
    p1 = points1_proj.reshape(NQ, C1).astype(f32)

    b0r, b1r = b0.reshape(1, 128), b1.reshape(1, 128)
    b2r, b3r = b2.reshape(1, 256), b3.reshape(1, 256)
    z2, st2 = _layer01(gmat, x1p, w0cat, w0a8, b0r,
                       g0.reshape(1, 128), be0.reshape(1, 128), W1, b1r)
    out = _layer234(z2, st2, g1.reshape(1, 128), be1.reshape(1, 128),
                    W2, b2r, g2.reshape(1, 256), be2.reshape(1, 256),
                    p1, W3[:256], W3[256:], b3r,
                    g3.reshape(1, 256), be3.reshape(1, 256))
    return out.reshape(B, H * W, 256)


# speedup vs baseline: 18.8936x; 1.0681x over previous
import functools

import jax
import jax.numpy as jnp
from jax import lax
from jax.experimental import pallas as pl
from jax.experimental.pallas import tpu as pltpu
from jax.experimental.pallas import tpu_sc as plsc

B, H, W, SH, SW = 2, 64, 256, 32, 128
KH, KW = 4, 8
NSAMPLE = 8
DIST = 100.0
C1, C2 = 64, 64
NQ = B * H * W
NTILES = 32
QT = NQ // NTILES
NGROUPS = QT // 16
TROWS = B * SH * SW
ZROW = TROWS
TPAD = 8200
GC = 128
EPS = 1e-5
NROWS = NSAMPLE * NQ


def _min_tree_kv(keys, idxs):
    keys = list(keys)
    idxs = list(idxs)
    while len(keys) > 1:
        nk, ni = [], []
        for i in range(len(keys) // 2):
            le = keys[2 * i] <= keys[2 * i + 1]
            nk.append(jnp.where(le, keys[2 * i], keys[2 * i + 1]))
            ni.append(jnp.where(le, idxs[2 * i], idxs[2 * i + 1]))
        if len(keys) % 2:
            nk.append(keys[-1])
            ni.append(idxs[-1])
        keys, idxs = nk, ni
    return keys[0], idxs[0]


def _sc_select_gather(x2flat, x1flatp, table):
    mesh = plsc.VectorSubcoreMesh(core_axis_name="c", subcore_axis_name="s")

    @functools.partial(
        pl.kernel,
        out_type=jax.ShapeDtypeStruct((NROWS, GC), jnp.float32),
        mesh=mesh,
        compiler_params=pltpu.CompilerParams(needs_layout_passes=False),
        scratch_types=[
            pltpu.VMEM((TPAD,), jnp.float32),
            pltpu.VMEM((TPAD,), jnp.float32),
            pltpu.VMEM((TPAD,), jnp.float32),
            pltpu.VMEM((QT,), jnp.float32),
            pltpu.VMEM((QT,), jnp.float32),
            pltpu.VMEM((QT,), jnp.float32),
            pltpu.VMEM((NSAMPLE * QT,), jnp.int32),
            pltpu.VMEM((256, GC), jnp.float32),
            pltpu.VMEM((256, GC), jnp.float32),
            pltpu.SemaphoreType.DMA,
            pltpu.SemaphoreType.DMA,
            pltpu.SemaphoreType.DMA,
            pltpu.SemaphoreType.DMA,
        ],
    )
    def body(x2f_h, x1f_h, tab_hbm, out_hbm,
             x2x, x2y, x2z, x1x, x1y, x1z, idxv, gbufa, gbufb,
             gsema, gsemb, osema, osemb):
        cid = lax.axis_index("c")
        sid = lax.axis_index("s")
        wid = sid * 2 + cid
        base = wid * QT
        pltpu.sync_copy(x2f_h.at[pl.ds(0, TPAD)], x2x)
        pltpu.sync_copy(x2f_h.at[pl.ds(TPAD, TPAD)], x2y)
        pltpu.sync_copy(x2f_h.at[pl.ds(2 * TPAD, TPAD)], x2z)
        pltpu.sync_copy(x1f_h.at[pl.ds(base, QT)], x1x)
        pltpu.sync_copy(x1f_h.at[pl.ds(NQ + base, QT)], x1y)
        pltpu.sync_copy(x1f_h.at[pl.ds(2 * NQ + base, QT)], x1z)

        lanes = lax.iota(jnp.int32, 16)
        sentinel = jnp.int32(0x7FFFFFFF)

        def g_body(g, carry):
            off = g * 16
            q = base + off + lanes
            wq = q & 255
            hq = (q >> 8) & 63
            bq = q >> 14
            h2 = hq >> 1
            w2 = wq >> 1
            x1 = x1x[pl.ds(off, 16)]
            y1 = x1y[pl.ds(off, 16)]
            z1 = x1z[pl.ds(off, 16)]
            zero16 = lanes - lanes
            keys = []
            kids = []
            for kh in range(KH):
                r = h2 + (kh - KH // 2)
                vr = (r >= 0) & (r < SH)
                rc = jnp.clip(r, 0, SH - 1)
                rbase = (bq << 12) + (rc << 7)
                for kw in range(KW):
                    c = w2 + (kw - KW // 2)
                    v = vr & (c >= 0) & (c < SW)
                    cc = jnp.clip(c, 0, SW - 1)
                    fl = rbase + cc
                    cx = plsc.load_gather(x2x, [fl])
                    cy = plsc.load_gather(x2y, [fl])
                    cz = plsc.load_gather(x2z, [fl])
                    dx = cx - x1
                    dy = cy - y1
                    dz = cz - z1
                    sx = plsc.bitcast(plsc.bitcast(dx * dx, jnp.int32),
                                      jnp.float32)
                    sy = plsc.bitcast(plsc.bitcast(dy * dy, jnp.int32),
                                      jnp.float32)
                    sz = plsc.bitcast(plsc.bitcast(dz * dz, jnp.int32),
                                      jnp.float32)
                    d2 = (sx + sy) + sz
                    keys.append(jnp.where(v, plsc.bitcast(d2, jnp.int32),
                                          sentinel))
                    kids.append(zero16 + (kh * KW + kw))
            for s in range(NSAMPLE):
                m, ksel = _min_tree_kv(keys, kids)
                d2s = plsc.bitcast(m, jnp.float32)
                ok = d2s < jnp.float32(DIST * DIST)
                khs = ksel >> 3
                kws = ksel & 7
                rs = jnp.clip(h2 + khs - KH // 2, 0, SH - 1)
                cs = jnp.clip(w2 + kws - KW // 2, 0, SW - 1)
                fl = (bq << 12) + (rs << 7) + cs
                idxv[pl.ds(s * QT + off, 16)] = jnp.where(ok, fl, jnp.int32(ZROW))
                if s < NSAMPLE - 1:
                    keys = [jnp.where(ksel == kk, sentinel, k)
                            for k, kk in zip(keys, range(32))]
            return carry

        lax.fori_loop(0, NGROUPS, g_body, 0)

        bufs = (gbufa, gbufb)
        gsems = (gsema, gsemb)
        osems = (osema, osemb)
        gd = [None, None]
        od = [None, None]
        for j in range(33):
            bsel = j & 1
            if j < 32:
                s, part = j >> 2, j & 3
                if od[bsel] is not None:
                    od[bsel].wait()
                    od[bsel] = None
                lo = s * QT + part * 256
                gd[bsel] = (
                    pltpu.async_copy(tab_hbm.at[idxv.at[pl.ds(lo, 128)]],
                                     bufs[bsel].at[pl.ds(0, 128)], gsems[bsel]),
                    pltpu.async_copy(tab_hbm.at[idxv.at[pl.ds(lo + 128, 128)]],
                                     bufs[bsel].at[pl.ds(128, 128)], gsems[bsel]),
                )
            pb = bsel ^ 1
            if gd[pb] is not None:
                gd[pb][0].wait()
                gd[pb][1].wait()
                gd[pb] = None
                jj = j - 1
                s2, part2 = jj >> 2, jj & 3
                od[pb] = pltpu.async_copy(
                    bufs[pb],
                    out_hbm.at[pl.ds(s2 * NQ + base + part2 * 256, 256)],
                    osems[pb])
        for pb in range(2):
            if od[pb] is not None:
                od[pb].wait()

    return body(x2flat, x1flatp, table)


def _stats_accum(st_ref, z, first):
    @pl.when(first)
    def _():
        st_ref[...] = jnp.zeros_like(st_ref)
    st_ref[0:1, :] += jnp.sum(z, axis=0, keepdims=True)
    st_ref[1:2, :] += jnp.sum(z * z, axis=0, keepdims=True)


def _bn_coeffs(st_ref, g_ref, be_ref, n):
    mu = st_ref[0:1, :] / n
    var = st_ref[1:2, :] / n - mu * mu
    inv = lax.rsqrt(var + EPS)
    scale = inv * g_ref[...]
    shift = be_ref[...] - mu * scale
    return scale, shift


def _z1_block(g_blk, x1_blk, wc_ref, wa_ref, b_ref):
    z = jnp.dot(g_blk, wc_ref[...], preferred_element_type=jnp.float32)
    z = z - jnp.dot(x1_blk, wa_ref[...], preferred_element_type=jnp.float32)
    return z + b_ref[...]


def _layer01(gmat, x1p, w0cat, w0a8, b0, g0, be0, W1m, b1):
    CQ = 2048
    NJ = NQ // CQ
    grid = (2, NSAMPLE, NJ)

    def body(g_ref, x1_ref, wc_ref, wa_ref, b_ref, g0_ref, be0_ref,
             w1_ref, b1_ref, z2_ref, st2_ref, st1v):
        p, i, j = pl.program_id(0), pl.program_id(1), pl.program_id(2)
        z = _z1_block(g_ref[0], x1_ref[...], wc_ref, wa_ref, b_ref)

        @pl.when(p == 0)
        def _():
            _stats_accum(st1v, z, (i == 0) & (j == 0))

        @pl.when(p == 1)
        def _():
            scale, shift = _bn_coeffs(st1v, g0_ref, be0_ref, float(NROWS))
            a = jnp.maximum(z * scale + shift, 0.0)
            z2 = jnp.dot(a, w1_ref[...], preferred_element_type=jnp.float32)
            z2 = z2 + b1_ref[...]
            z2_ref[0] = z2.astype(jnp.bfloat16)
            _stats_accum(st2_ref, z2, (i == 0) & (j == 0))

    return pl.pallas_call(
        body,
        grid=grid,
        in_specs=[
            pl.BlockSpec((1, CQ, GC), lambda p, i, j: (i, j, 0)),
            pl.BlockSpec((CQ, 8), lambda p, i, j: (j, 0)),
            pl.BlockSpec((GC, 128), lambda p, i, j: (0, 0)),
            pl.BlockSpec((8, 128), lambda p, i, j: (0, 0)),
            pl.BlockSpec((1, 128), lambda p, i, j: (0, 0)),
            pl.BlockSpec((1, 128), lambda p, i, j: (0, 0)),
            pl.BlockSpec((1, 128), lambda p, i, j: (0, 0)),
            pl.BlockSpec((128, 128), lambda p, i, j: (0, 0)),
            pl.BlockSpec((1, 128), lambda p, i, j: (0, 0)),
        ],
        out_specs=[
            pl.BlockSpec((1, CQ, 128),
                         lambda p, i, j: (jnp.where(p == 0, NSAMPLE, i),
                                          jnp.where(p == 0, 0, j), 0)),
            pl.BlockSpec((8, 128), lambda p, i, j: (0, 0)),
        ],
        out_shape=[
            jax.ShapeDtypeStruct((NSAMPLE + 1, NQ, 128), jnp.bfloat16),
            jax.ShapeDtypeStruct((8, 128), jnp.float32),
        ],
        scratch_shapes=[pltpu.VMEM((8, 128), jnp.float32)],
    )(gmat.reshape(NSAMPLE, NQ, GC), x1p, w0cat, w0a8, b0, g0, be0, W1m, b1)


def _layer234(z2, st2, g1, be1, W2m, b2, g2, be2, p1, w3m, w3p, b3, g3, be3):
    CQ = 1024
    NJ = NQ // CQ
    grid = (3, NJ)

    def body(z_ref, st_ref, g_ref, be_ref, w2_ref, b2_ref, g2_ref,
             be2_ref, p_ref, wm_ref, wp_ref, b_ref, g3_ref, be3_ref,
             o_ref, st3v, st4v, z4v):
        p, i = pl.program_id(0), pl.program_id(1)

        @pl.when(p < 2)
        def _():
            scale, shift = _bn_coeffs(st_ref, g_ref, be_ref, float(NROWS))
            a = jnp.maximum(z_ref[...].astype(jnp.float32) * scale + shift, 0.0)
            a = a.reshape(NSAMPLE * CQ, 128)
            z3 = jnp.dot(a, w2_ref[...],
                         preferred_element_type=jnp.float32) + b2_ref[...]

            @pl.when(p == 0)
            def _():
                _stats_accum(st3v, z3, i == 0)

            @pl.when(p == 1)
            def _():
                scale3, shift3 = _bn_coeffs(st3v, g2_ref, be2_ref, float(NROWS))
                a3 = jnp.maximum(z3 * scale3 + shift3,
                                 0.0).reshape(NSAMPLE, CQ, 256)
                m = jnp.max(a3, axis=0)
                z4 = jnp.dot(m, wm_ref[...], preferred_element_type=jnp.float32)
                z4 = z4 + jnp.dot(p_ref[...], wp_ref[...],
                                  preferred_element_type=jnp.float32)
                z4 = z4 + b_ref[...]
                z4v[i] = z4
                _stats_accum(st4v, z4, i == 0)

        @pl.when(p == 2)
        def _():
            scale4, shift4 = _bn_coeffs(st4v, g3_ref, be3_ref, float(NQ))
            o_ref[...] = jnp.maximum(z4v[i] * scale4 + shift4, 0.0)

    return pl.pallas_call(
        body,
        grid=grid,
        in_specs=[
            pl.BlockSpec((NSAMPLE, CQ, 128),
                         lambda p, i: (0, jnp.where(p == 2, 0, i), 0)),
            pl.BlockSpec((8, 128), lambda p, i: (0, 0)),
            pl.BlockSpec((1, 128), lambda p, i: (0, 0)),
            pl.BlockSpec((1, 128), lambda p, i: (0, 0)),
            pl.BlockSpec((128, 256), lambda p, i: (0, 0)),
            pl.BlockSpec((1, 256), lambda p, i: (0, 0)),
            pl.BlockSpec((1, 256), lambda p, i: (0, 0)),
            pl.BlockSpec((1, 256), lambda p, i: (0, 0)),
            pl.BlockSpec((CQ, C1), lambda p, i: (jnp.where(p == 2, 0, i), 0)),
            pl.BlockSpec((256, 256), lambda p, i: (0, 0)),
            pl.BlockSpec((C1, 256), lambda p, i: (0, 0)),
            pl.BlockSpec((1, 256), lambda p, i: (0, 0)),
            pl.BlockSpec((1, 256), lambda p, i: (0, 0)),
            pl.BlockSpec((1, 256), lambda p, i: (0, 0)),
        ],
        out_specs=pl.BlockSpec((CQ, 256),
                               lambda p, i: (jnp.where(p == 2, i, 0), 0)),
        out_shape=jax.ShapeDtypeStruct((NQ, 256), jnp.float32),
        scratch_shapes=[
            pltpu.VMEM((8, 256), jnp.float32),
            pltpu.VMEM((8, 256), jnp.float32),
            pltpu.VMEM((NJ, CQ, 256), jnp.float32),
        ],
    )(z2, st2, g1, be1, W2m, b2, g2, be2, p1, w3m, w3p, b3, g3, be3)


def kernel(xyz1_proj, xyz2_proj, points1_proj, feat2_proj,
           W0, b0, g0, be0, W1, b1, g1, be1, W2, b2, g2, be2,
           W3, b3, g3, be3):
    f32 = jnp.float32
    t_xyz = xyz2_proj.reshape(TROWS, 3).astype(f32)
    t_feat = feat2_proj.reshape(TROWS, C2).astype(f32)
    table = jnp.pad(jnp.concatenate([t_xyz, t_feat], axis=1),
                    ((0, TPAD - TROWS), (0, GC - 3 - C2)))
    x2flat = jnp.pad(t_xyz.T, ((0, 0), (0, TPAD - TROWS))).reshape(-1)
    x1flat = xyz1_proj.reshape(NQ, 3).astype(f32)
    x1flatp = x1flat.T.reshape(-1)

    gmat = _sc_select_gather(x2flat, x1flatp, table)

    w0cat = jnp.pad(W0, ((0, GC - 3 - C2), (0, 0)))
    w0a8 = jnp.pad(W0[:3], ((0, 5), (0, 0)))
    x1p = jnp.pad(x1flat, ((0, 0), (0, 5)))
    p1 = points1_proj.reshape(NQ, C1).astype(f32)

    b0r, b1r = b0.reshape(1, 128), b1.reshape(1, 128)
    b2r, b3r = b2.reshape(1, 256), b3.reshape(1, 256)
    z2, st2 = _layer01(gmat, x1p, w0cat, w0a8, b0r,
                       g0.reshape(1, 128), be0.reshape(1, 128), W1, b1r)
    out = _layer234(z2, st2, g1.reshape(1, 128), be1.reshape(1, 128),
                    W2, b2r, g2.reshape(1, 256), be2.reshape(1, 256),
                    p1, W3[:256], W3[256:], b3r,
                    g3.reshape(1, 256), be3.reshape(1, 256))
    return out.reshape(B, H * W, 256)
